# Initial kernel scaffold; baseline (speedup 1.0000x reference)
import jax
import jax.numpy as jnp
from jax.experimental import pallas as pl


def kernel(x, edge_index, W_in, b_in, W_gat0, att_src0, att_dst0, b_gat0, W_gat1, att_src1, att_dst1, b_gat1, W_gat2, att_src2, att_dst2, b_gat2, W_out1, b_out1, W_out2, b_out2):
    raise NotImplementedError("write your pallas kernel here")


# speedup vs baseline: 22.5046x; 22.5046x over previous
import functools

import jax
import jax.numpy as jnp
from jax import lax
from jax.experimental import pallas as pl
from jax.experimental.pallas import tpu as pltpu
from jax.experimental.pallas import tpu_sc as plsc

N = 10000
E = 320000
F_IN = 128
H = 64

NC = 2
NS = 16
LANES = 16

NP = 10112
ROWW = 48
DUMMY = N

PT = 20480
EP = NS * PT
CH = 512
NCHUNK = PT // CH
IDXR = CH // 128
GROUPS = CH // LANES
ROWS_PER_TILE = NP // NS

_HIGH = jax.lax.Precision.HIGHEST


def _lrelu(t):
    return jnp.where(t > 0, t, 0.2 * t)


BLK = NP // 8


def _dot(a, b):
    return jax.lax.dot_general(a, b, (((1,), (0,)), ((), ())),
                               precision=_HIGH,
                               preferred_element_type=jnp.float32)


def _attn_tail(h, asv, adv, table_ref, asrc_ref, adst_ref):
    asrc_ref[...] = _dot(h, asv)
    adst_ref[...] = _dot(h, adv)
    ones = jnp.ones((BLK, 1), jnp.float32)
    zeros = jnp.zeros((BLK, ROWW - 17), jnp.float32)
    table_ref[0] = h[:, :ROWW]
    table_ref[1] = jnp.concatenate([h[:, ROWW:], ones, zeros], axis=1)


def _tc_pre_body(x_ref, w_in_ref, b_in_ref, w_g_ref, asv_ref, adv_ref,
                 table_ref, asrc_ref, adst_ref):
    x1 = jnp.maximum(_dot(x_ref[...], w_in_ref[...]) + b_in_ref[...], 0.0)
    h = _dot(x1, w_g_ref[...])
    _attn_tail(h, asv_ref[...], adv_ref[...], table_ref, asrc_ref, adst_ref)


def _combine(acc_ref, tprev_ref, exs_ref, b_ref):
    acc0 = acc_ref[0]
    acc1 = acc_ref[1]
    hprev = jnp.concatenate(
        [tprev_ref[0], tprev_ref[1][:, :H - ROWW]], axis=1)
    exs = exs_ref[...]
    num = jnp.concatenate([acc0, acc1[:, :H - ROWW]], axis=1) + exs * hprev
    den = acc1[:, H - ROWW:H - ROWW + 1] + exs + 1e-16
    return num / den + b_ref[...]


def _tc_mid_body(acc_ref, tprev_ref, exs_ref, b_ref, w_g_ref, asv_ref,
                 adv_ref, table_ref, asrc_ref, adst_ref):
    y = jnp.maximum(_combine(acc_ref, tprev_ref, exs_ref, b_ref), 0.0)
    h = _dot(y, w_g_ref[...])
    _attn_tail(h, asv_ref[...], adv_ref[...], table_ref, asrc_ref, adst_ref)


def _tc_shift_body(asrc_ref, adst_ref, cvec_ref, exs_ref):
    asrc = asrc_ref[...]
    adst = adst_ref[...]
    rows = jax.lax.broadcasted_iota(jnp.int32, (NP, 1), 0)
    mask = rows < N
    neg = jnp.float32(-1e30)
    ms = jnp.max(jnp.where(mask, asrc, neg))
    md = jnp.max(jnp.where(mask, adst, neg))
    c = _lrelu(ms + md)
    cvec_ref[...] = jnp.full((1, 128), c, jnp.float32)
    exs_ref[...] = jnp.exp(_lrelu(asrc + adst) - c)


def _tc_post_body(acc_ref, tprev_ref, exs_ref, b_ref, w1_ref, b1_ref,
                  w2_ref, b2_ref, out_ref):
    y = _combine(acc_ref, tprev_ref, exs_ref, b_ref)
    z = jnp.maximum(_dot(y, w1_ref[...]) + b1_ref[...], 0.0)
    out_ref[...] = _dot(z, w2_ref[...]) + b2_ref[...]


def _full(shape):
    return pl.BlockSpec(shape, lambda i: tuple(0 for _ in shape))


_ROWBLK = pl.BlockSpec((BLK, None), lambda i: (i, 0))
_ROW1 = pl.BlockSpec((BLK, 1), lambda i: (i, 0))
_ACCBLK = pl.BlockSpec((NC, BLK, ROWW), lambda i: (0, i, 0))

_ATTN_OUTS = (
    jax.ShapeDtypeStruct((NC, NP, ROWW), jnp.float32),
    jax.ShapeDtypeStruct((NP, 1), jnp.float32),
    jax.ShapeDtypeStruct((NP, 1), jnp.float32),
)
_ATTN_OUT_SPECS = (_ACCBLK, _ROW1, _ROW1)

_tc_pre = pl.pallas_call(
    _tc_pre_body, out_shape=_ATTN_OUTS, grid=(NP // BLK,),
    in_specs=[pl.BlockSpec((BLK, F_IN), lambda i: (i, 0)),
              _full((F_IN, H)), _full((1, H)), _full((H, H)),
              _full((H, 1)), _full((H, 1))],
    out_specs=_ATTN_OUT_SPECS)
_tc_mid = pl.pallas_call(
    _tc_mid_body, out_shape=_ATTN_OUTS, grid=(NP // BLK,),
    in_specs=[_ACCBLK, _ACCBLK, _ROW1, _full((1, H)), _full((H, H)),
              _full((H, 1)), _full((H, 1))],
    out_specs=_ATTN_OUT_SPECS)
_tc_shift = pl.pallas_call(
    _tc_shift_body,
    out_shape=(jax.ShapeDtypeStruct((1, 128), jnp.float32),
               jax.ShapeDtypeStruct((NP, 1), jnp.float32)))
_tc_post = pl.pallas_call(
    _tc_post_body, out_shape=jax.ShapeDtypeStruct((NP, 32), jnp.float32),
    grid=(NP // BLK,),
    in_specs=[_ACCBLK, _ACCBLK, _ROW1, _full((1, H)), _full((H, H)),
              _full((1, H)), _full((H, 32)), _full((1, 32))],
    out_specs=pl.BlockSpec((BLK, 32), lambda i: (i, 0)))


def _sc_edge_body(src_hbm, dst_hbm, asrc_hbm, adst_hbm, table_hbm, cvec_hbm,
                  acc_hbm,
                  asrc_v, adst_v, cvec_v, sidx_v, didx_v, ex_v, rows_v,
                  out_v, acc_sh, sem):
    c = lax.axis_index("c")
    s = lax.axis_index("s")
    coff = jnp.full((LANES,), c * NP, jnp.int32)

    pltpu.sync_copy(asrc_hbm, asrc_v)
    pltpu.sync_copy(adst_hbm, adst_v)
    pltpu.sync_copy(cvec_hbm, cvec_v)
    cval = cvec_v[pl.ds(0, LANES)]

    zero16 = jnp.zeros((LANES,), jnp.float32)

    def _zero_rows(e, _):
        for k in range(ROWW // LANES):
            rows_v[e, pl.ds(k * LANES, LANES)] = zero16
        return 0

    lax.fori_loop(0, CH, _zero_rows, 0)
    base = s * ROWS_PER_TILE
    pltpu.sync_copy(rows_v.at[pl.ds(0, CH)], acc_sh.at[pl.ds(base, CH)])
    pltpu.sync_copy(rows_v.at[pl.ds(0, ROWS_PER_TILE - CH)],
                    acc_sh.at[pl.ds(base + CH, ROWS_PER_TILE - CH)])
    plsc.subcore_barrier()

    def _chunk(ch, _):
        idx_row = s * (PT // 128) + ch * IDXR
        pltpu.sync_copy(src_hbm.at[pl.ds(idx_row, IDXR)], sidx_v)
        pltpu.sync_copy(dst_hbm.at[pl.ds(idx_row, IDXR)], didx_v)

        def _alpha(g, _):
            jj, gg = g // 8, (g % 8) * LANES
            sv = sidx_v[jj, pl.ds(gg, LANES)]
            dv = didx_v[jj, pl.ds(gg, LANES)]
            a1 = plsc.load_gather(asrc_v, [sv])
            a2 = plsc.load_gather(adst_v, [dv])
            t = a1 + a2
            ex = jnp.exp(_lrelu(t) - cval)
            ex_v[pl.ds(g * LANES, LANES)] = ex
            sidx_v[jj, pl.ds(gg, LANES)] = sv + coff
            return 0

        lax.fori_loop(0, GROUPS, _alpha, 0, unroll=2)

        cps = []
        for j in range(IDXR):
            cps.append(pltpu.async_copy(
                table_hbm.at[sidx_v.at[j]],
                rows_v.at[pl.ds(j * 128, 128)], sem))
        for cp in cps:
            cp.wait()

        def _scale(g, _):
            exv = ex_v[pl.ds(g * LANES, LANES)]
            for i in range(LANES):
                bi = exv.at[jnp.full((LANES,), i, jnp.int32)].get(
                    mode="promise_in_bounds")
                e = g * LANES + i
                for k in range(ROWW // LANES):
                    sl = pl.ds(k * LANES, LANES)
                    rows_v[e, sl] = rows_v[e, sl] * bi
            return 0

        lax.fori_loop(0, GROUPS, _scale, 0)

        for j in range(IDXR):
            pltpu.sync_copy(rows_v.at[pl.ds(j * 128, 128)],
                            acc_sh.at[didx_v.at[j]], add=True)
        return 0

    lax.fori_loop(0, NCHUNK, _chunk, 0)
    plsc.subcore_barrier()

    pltpu.sync_copy(acc_sh.at[pl.ds(base, ROWS_PER_TILE)], out_v)
    pltpu.sync_copy(out_v, acc_hbm.at[c].at[pl.ds(base, ROWS_PER_TILE)])


@functools.cache
def _get_sc_edge():
  return pl.kernel(
    _sc_edge_body,
    out_type=jax.ShapeDtypeStruct((NC, NP, ROWW), jnp.float32),
    mesh=plsc.VectorSubcoreMesh(core_axis_name="c", subcore_axis_name="s",
                                num_cores=NC, num_subcores=NS),
    compiler_params=pltpu.CompilerParams(needs_layout_passes=False,
                                         use_tc_tiling_on_sc=False),
    scratch_types=[
        pltpu.VMEM((NP,), jnp.float32),
        pltpu.VMEM((NP,), jnp.float32),
        pltpu.VMEM((128,), jnp.float32),
        pltpu.VMEM((IDXR, 128), jnp.int32),
        pltpu.VMEM((IDXR, 128), jnp.int32),
        pltpu.VMEM((CH,), jnp.float32),
        pltpu.VMEM((CH, ROWW), jnp.float32),
        pltpu.VMEM((ROWS_PER_TILE, ROWW), jnp.float32),
        pltpu.VMEM_SHARED((NP, ROWW), jnp.float32),
        pltpu.SemaphoreType.DMA,
    ],
  )


def kernel(x, edge_index, W_in, b_in,
           W_gat0, att_src0, att_dst0, b_gat0,
           W_gat1, att_src1, att_dst1, b_gat1,
           W_gat2, att_src2, att_dst2, b_gat2,
           W_out1, b_out1, W_out2, b_out2):
    f32 = jnp.float32
    x_p = jnp.pad(x.astype(f32), ((0, NP - N), (0, 0)))
    pad = jnp.full((EP - E,), DUMMY, jnp.int32)
    src2 = jnp.concatenate([edge_index[0].astype(jnp.int32), pad]
                           ).reshape(EP // 128, 128)
    dst2 = jnp.concatenate([edge_index[1].astype(jnp.int32), pad]
                           ).reshape(EP // 128, 128)

    asv = [a.reshape(H, 1).astype(f32) for a in (att_src0, att_src1, att_src2)]
    adv = [a.reshape(H, 1).astype(f32) for a in (att_dst0, att_dst1, att_dst2)]
    Wg = [W_gat0.astype(f32), W_gat1.astype(f32), W_gat2.astype(f32)]
    bg = [b_gat0.astype(f32), b_gat1.astype(f32), b_gat2.astype(f32)]

    table, asrc, adst = _tc_pre(
        x_p, W_in.astype(f32), b_in.astype(f32).reshape(1, H), Wg[0],
        asv[0], adv[0])
    for l in range(3):
        cvec, exs = _tc_shift(asrc, adst)
        acc = _get_sc_edge()(src2, dst2,
                             asrc.reshape(NP), adst.reshape(NP),
                             table.reshape(NC * NP, ROWW),
                             cvec.reshape(128))
        if l < 2:
            table, asrc, adst = _tc_mid(
                acc, table, exs, bg[l].reshape(1, H), Wg[l + 1],
                asv[l + 1], adv[l + 1])
        else:
            out = _tc_post(acc, table, exs, bg[l].reshape(1, H),
                           W_out1.astype(f32), b_out1.astype(f32).reshape(1, H),
                           W_out2.astype(f32), b_out2.astype(f32).reshape(1, 32))
    return out[:N]


# speedup vs baseline: 30.5557x; 1.3578x over previous
import functools

import jax
import jax.numpy as jnp
from jax import lax
from jax.experimental import pallas as pl
from jax.experimental.pallas import tpu as pltpu
from jax.experimental.pallas import tpu_sc as plsc

N = 10000
E = 320000
F_IN = 128
H = 64

NC = 2
NS = 16
LANES = 16

NP = 10112
ROWW = 32
DUMMY = N
NPD_R = 640
NPD_C = 16

PT = 20480
EP = NS * PT
CH = 512
NCHUNK = PT // CH
IDXR = CH // 128
GROUPS = CH // LANES
ROWS_PER_TILE = NP // NS

_HIGH = jax.lax.Precision.HIGHEST


def _lrelu(t):
    return jnp.where(t > 0, t, 0.2 * t)


BLK = NP // 8


def _dot(a, b):
    return jax.lax.dot_general(a, b, (((1,), (0,)), ((), ())),
                               precision=_HIGH,
                               preferred_element_type=jnp.float32)


def _attn_tail(h, asv, adv, table_ref, asrc_ref, adst_ref):
    asrc_ref[...] = _dot(h, asv)
    adst_ref[...] = _dot(h, adv)
    table_ref[0] = h[:, :ROWW]
    table_ref[1] = h[:, ROWW:]


def _tc_pre_body(x_ref, w_in_ref, b_in_ref, w_g_ref, asv_ref, adv_ref,
                 table_ref, asrc_ref, adst_ref):
    x1 = jnp.maximum(_dot(x_ref[...], w_in_ref[...]) + b_in_ref[...], 0.0)
    h = _dot(x1, w_g_ref[...])
    _attn_tail(h, asv_ref[...], adv_ref[...], table_ref, asrc_ref, adst_ref)


def _combine(acc_ref, tprev_ref, den_ref, exs_ref, b_ref):
    hprev = jnp.concatenate([tprev_ref[0], tprev_ref[1]], axis=1)
    exs = exs_ref[...]
    num = (jnp.concatenate([acc_ref[0], acc_ref[1]], axis=1)
           + exs * hprev)
    den = den_ref[...] + exs + 1e-16
    return num / den + b_ref[...]


def _tc_mid_body(acc_ref, tprev_ref, den_ref, exs_ref, b_ref, w_g_ref,
                 asv_ref, adv_ref, table_ref, asrc_ref, adst_ref):
    y = jnp.maximum(_combine(acc_ref, tprev_ref, den_ref, exs_ref, b_ref),
                    0.0)
    h = _dot(y, w_g_ref[...])
    _attn_tail(h, asv_ref[...], adv_ref[...], table_ref, asrc_ref, adst_ref)


def _tc_shift_body(asrc_ref, adst_ref, cvec_ref, exs_ref):
    asrc = asrc_ref[...]
    adst = adst_ref[...]
    rows = jax.lax.broadcasted_iota(jnp.int32, (NP, 1), 0)
    mask = rows < N
    neg = jnp.float32(-1e30)
    ms = jnp.max(jnp.where(mask, asrc, neg))
    md = jnp.max(jnp.where(mask, adst, neg))
    c = _lrelu(ms + md)
    cvec_ref[...] = jnp.full((1, 128), c, jnp.float32)
    exs_ref[...] = jnp.exp(_lrelu(asrc + adst) - c)


def _tc_post_body(acc_ref, tprev_ref, den_ref, exs_ref, b_ref, w1_ref,
                  b1_ref, w2_ref, b2_ref, out_ref):
    y = _combine(acc_ref, tprev_ref, den_ref, exs_ref, b_ref)
    z = jnp.maximum(_dot(y, w1_ref[...]) + b1_ref[...], 0.0)
    out_ref[...] = _dot(z, w2_ref[...]) + b2_ref[...]


def _full(shape):
    return pl.BlockSpec(shape, lambda i: tuple(0 for _ in shape))


_ROWBLK = pl.BlockSpec((BLK, None), lambda i: (i, 0))
_ROW1 = pl.BlockSpec((BLK, 1), lambda i: (i, 0))
_ACCBLK = pl.BlockSpec((NC, BLK, ROWW), lambda i: (0, i, 0))

_ATTN_OUTS = (
    jax.ShapeDtypeStruct((NC, NP, ROWW), jnp.float32),
    jax.ShapeDtypeStruct((NP, 1), jnp.float32),
    jax.ShapeDtypeStruct((NP, 1), jnp.float32),
)
_ATTN_OUT_SPECS = (_ACCBLK, _ROW1, _ROW1)

_tc_pre = pl.pallas_call(
    _tc_pre_body, out_shape=_ATTN_OUTS, grid=(NP // BLK,),
    in_specs=[pl.BlockSpec((BLK, F_IN), lambda i: (i, 0)),
              _full((F_IN, H)), _full((1, H)), _full((H, H)),
              _full((H, 1)), _full((H, 1))],
    out_specs=_ATTN_OUT_SPECS)
_tc_mid = pl.pallas_call(
    _tc_mid_body, out_shape=_ATTN_OUTS, grid=(NP // BLK,),
    in_specs=[_ACCBLK, _ACCBLK, _ROW1, _ROW1, _full((1, H)), _full((H, H)),
              _full((H, 1)), _full((H, 1))],
    out_specs=_ATTN_OUT_SPECS)
_tc_shift = pl.pallas_call(
    _tc_shift_body,
    out_shape=(jax.ShapeDtypeStruct((1, 128), jnp.float32),
               jax.ShapeDtypeStruct((NP, 1), jnp.float32)))
_tc_post = pl.pallas_call(
    _tc_post_body, out_shape=jax.ShapeDtypeStruct((NP, 32), jnp.float32),
    grid=(NP // BLK,),
    in_specs=[_ACCBLK, _ACCBLK, _ROW1, _ROW1, _full((1, H)), _full((H, H)),
              _full((1, H)), _full((H, 32)), _full((1, 32))],
    out_specs=pl.BlockSpec((BLK, 32), lambda i: (i, 0)))


def _sc_edge_body(src_hbm, dst_hbm, asrc_hbm, adst_hbm, table_hbm, cvec_hbm,
                  acc_hbm, den_hbm,
                  asrc_v, adst_v, cvec_v,
                  sidx_a, sidx_b, didx_a, didx_b, drow_a, drow_b,
                  ex_a, ex_b, exd_a, exd_b,
                  rows_a, rows_b, out_v, dbuf_v, acc_sh, den_sh,
                  sem_ia, sem_ib, sem_ga, sem_gb):
    c = lax.axis_index("c")
    s = lax.axis_index("s")
    coff = jnp.full((LANES,), c * NP, jnp.int32)
    iota = lax.iota(jnp.int32, LANES)
    zero16 = jnp.zeros((LANES,), jnp.float32)
    on_core0 = c == 0

    sidx = (sidx_a, sidx_b)
    didx = (didx_a, didx_b)
    drow = (drow_a, drow_b)
    exv_ = (ex_a, ex_b)
    exd = (exd_a, exd_b)
    rows = (rows_a, rows_b)
    sem_i = (sem_ia, sem_ib)
    sem_g = (sem_ga, sem_gb)

    pltpu.sync_copy(asrc_hbm, asrc_v)
    pltpu.sync_copy(adst_hbm, adst_v)
    pltpu.sync_copy(cvec_hbm, cvec_v)
    cval = cvec_v[pl.ds(0, LANES)]

    def _zero_rows(e, _):
        for k in range(ROWW // LANES):
            rows_a[e, pl.ds(k * LANES, LANES)] = zero16
        exd_a[e, pl.ds(0, LANES)] = zero16
        exd_b[e, pl.ds(0, LANES)] = zero16
        return 0

    lax.fori_loop(0, CH, _zero_rows, 0)
    base = s * ROWS_PER_TILE
    pltpu.sync_copy(rows_a.at[pl.ds(0, CH)], acc_sh.at[pl.ds(base, CH)])
    pltpu.sync_copy(rows_a.at[pl.ds(0, ROWS_PER_TILE - CH)],
                    acc_sh.at[pl.ds(base + CH, ROWS_PER_TILE - CH)])

    @pl.when(on_core0)
    def _():
        def _zero_dbuf(e, _):
            dbuf_v[e, pl.ds(0, LANES)] = zero16
            return 0
        lax.fori_loop(0, NPD_R // NS, _zero_dbuf, 0)
        pltpu.sync_copy(dbuf_v, den_sh.at[pl.ds(s * (NPD_R // NS),
                                                NPD_R // NS)])

    plsc.subcore_barrier()

    idx_base = s * (PT // 128)

    def _issue_idx(k, b):
        pltpu.async_copy(src_hbm.at[pl.ds(idx_base + k * IDXR, IDXR)],
                         sidx[b], sem_i[b])
        pltpu.async_copy(dst_hbm.at[pl.ds(idx_base + k * IDXR, IDXR)],
                         didx[b], sem_i[b])

    def _wait_idx(b):
        pltpu.make_async_copy(src_hbm.at[pl.ds(0, IDXR)], sidx[b],
                              sem_i[b]).wait()
        pltpu.make_async_copy(dst_hbm.at[pl.ds(0, IDXR)], didx[b],
                              sem_i[b]).wait()

    def _alpha_chunk(b):
        def _alpha(g, _):
            jj, gg = g // 8, (g % 8) * LANES
            sv = sidx[b][jj, pl.ds(gg, LANES)]
            dv = didx[b][jj, pl.ds(gg, LANES)]
            a1 = plsc.load_gather(asrc_v, [sv])
            a2 = plsc.load_gather(adst_v, [dv])
            t = a1 + a2
            ex = jnp.exp(_lrelu(t) - cval)
            exv_[b][pl.ds(g * LANES, LANES)] = ex
            sidx[b][jj, pl.ds(gg, LANES)] = sv + coff

            @pl.when(on_core0)
            def _():
                drow[b][jj, pl.ds(gg, LANES)] = (
                    lax.shift_right_logical(dv, 4))
                ev = g * LANES + iota
                plsc.store_scatter(exd[b], [ev, jnp.bitwise_and(dv, 15)], ex)
            return 0

        lax.fori_loop(0, GROUPS, _alpha, 0, unroll=2)

    def _issue_gather(b):
        for j in range(IDXR):
            pltpu.async_copy(table_hbm.at[sidx[b].at[j]],
                             rows[b].at[pl.ds(j * 128, 128)], sem_g[b])

    def _wait_gather(b):
        pltpu.make_async_copy(table_hbm.at[pl.ds(0, CH)], rows[b],
                              sem_g[b]).wait()

    def _scale_scatter(b):
        def _scale(g, _):
            exv = exv_[b][pl.ds(g * LANES, LANES)]
            for i in range(LANES):
                bi = exv.at[jnp.full((LANES,), i, jnp.int32)].get(
                    mode="promise_in_bounds")
                e = g * LANES + i
                for k in range(ROWW // LANES):
                    sl = pl.ds(k * LANES, LANES)
                    rows[b][e, sl] = rows[b][e, sl] * bi
            return 0

        lax.fori_loop(0, GROUPS, _scale, 0, unroll=2)
        for j in range(IDXR):
            pltpu.sync_copy(rows[b].at[pl.ds(j * 128, 128)],
                            acc_sh.at[didx[b].at[j]], add=True)

        @pl.when(on_core0)
        def _():
            for j in range(IDXR):
                pltpu.sync_copy(exd[b].at[pl.ds(j * 128, 128)],
                                den_sh.at[drow[b].at[j]], add=True)

            def _rezero(g, _):
                jj, gg = g // 8, (g % 8) * LANES
                dv = didx[b][jj, pl.ds(gg, LANES)]
                ev = g * LANES + iota
                plsc.store_scatter(exd[b], [ev, jnp.bitwise_and(dv, 15)],
                                   zero16)
                return 0

            lax.fori_loop(0, GROUPS, _rezero, 0, unroll=2)

    _issue_idx(0, 0)

    def _pair(base_k, _):
        for b in range(2):
            k = base_k + b
            _wait_idx(b)
            _alpha_chunk(b)
            _issue_gather(b)

            @pl.when(k > 0)
            def _():
                _wait_gather(1 - b)
                _scale_scatter(1 - b)

            @pl.when(k + 1 < NCHUNK)
            def _():
                _issue_idx(k + 1, 1 - b)
        return 0

    lax.fori_loop(0, NCHUNK // 2, lambda i, u: _pair(i * 2, u), 0)
    _wait_gather(1)
    _scale_scatter(1)
    plsc.subcore_barrier()

    pltpu.sync_copy(acc_sh.at[pl.ds(base, ROWS_PER_TILE)], out_v)
    pltpu.sync_copy(out_v, acc_hbm.at[c].at[pl.ds(base, ROWS_PER_TILE)])

    @pl.when(on_core0)
    def _():
        dslice = pl.ds(s * (NPD_R // NS), NPD_R // NS)
        pltpu.sync_copy(den_sh.at[dslice], dbuf_v)
        pltpu.sync_copy(dbuf_v, den_hbm.at[dslice])


@functools.cache
def _get_sc_edge():
  return pl.kernel(
    _sc_edge_body,
    out_type=(jax.ShapeDtypeStruct((NC, NP, ROWW), jnp.float32),
              jax.ShapeDtypeStruct((NPD_R, NPD_C), jnp.float32)),
    mesh=plsc.VectorSubcoreMesh(core_axis_name="c", subcore_axis_name="s",
                                num_cores=NC, num_subcores=NS),
    compiler_params=pltpu.CompilerParams(needs_layout_passes=False,
                                         use_tc_tiling_on_sc=False),
    scratch_types=[
        pltpu.VMEM((NP,), jnp.float32),
        pltpu.VMEM((NP,), jnp.float32),
        pltpu.VMEM((128,), jnp.float32),
        pltpu.VMEM((IDXR, 128), jnp.int32),
        pltpu.VMEM((IDXR, 128), jnp.int32),
        pltpu.VMEM((IDXR, 128), jnp.int32),
        pltpu.VMEM((IDXR, 128), jnp.int32),
        pltpu.VMEM((IDXR, 128), jnp.int32),
        pltpu.VMEM((IDXR, 128), jnp.int32),
        pltpu.VMEM((CH,), jnp.float32),
        pltpu.VMEM((CH,), jnp.float32),
        pltpu.VMEM((CH, NPD_C), jnp.float32),
        pltpu.VMEM((CH, NPD_C), jnp.float32),
        pltpu.VMEM((CH, ROWW), jnp.float32),
        pltpu.VMEM((CH, ROWW), jnp.float32),
        pltpu.VMEM((ROWS_PER_TILE, ROWW), jnp.float32),
        pltpu.VMEM((NPD_R // NS, NPD_C), jnp.float32),
        pltpu.VMEM_SHARED((NP, ROWW), jnp.float32),
        pltpu.VMEM_SHARED((NPD_R, NPD_C), jnp.float32),
        pltpu.SemaphoreType.DMA,
        pltpu.SemaphoreType.DMA,
        pltpu.SemaphoreType.DMA,
        pltpu.SemaphoreType.DMA,
    ],
  )


def kernel(x, edge_index, W_in, b_in,
           W_gat0, att_src0, att_dst0, b_gat0,
           W_gat1, att_src1, att_dst1, b_gat1,
           W_gat2, att_src2, att_dst2, b_gat2,
           W_out1, b_out1, W_out2, b_out2):
    f32 = jnp.float32
    x_p = jnp.pad(x.astype(f32), ((0, NP - N), (0, 0)))
    pad = jnp.full((EP - E,), DUMMY, jnp.int32)
    src2 = jnp.concatenate([edge_index[0].astype(jnp.int32), pad]
                           ).reshape(EP // 128, 128)
    dst2 = jnp.concatenate([edge_index[1].astype(jnp.int32), pad]
                           ).reshape(EP // 128, 128)

    asv = [a.reshape(H, 1).astype(f32) for a in (att_src0, att_src1, att_src2)]
    adv = [a.reshape(H, 1).astype(f32) for a in (att_dst0, att_dst1, att_dst2)]
    Wg = [W_gat0.astype(f32), W_gat1.astype(f32), W_gat2.astype(f32)]
    bg = [b_gat0.astype(f32), b_gat1.astype(f32), b_gat2.astype(f32)]

    table, asrc, adst = _tc_pre(
        x_p, W_in.astype(f32), b_in.astype(f32).reshape(1, H), Wg[0],
        asv[0], adv[0])
    for l in range(3):
        cvec, exs = _tc_shift(asrc, adst)
        acc, den = _get_sc_edge()(src2, dst2,
                                  asrc.reshape(NP), adst.reshape(NP),
                                  table.reshape(NC * NP, ROWW),
                                  cvec.reshape(128))
        den = den.reshape(NPD_R * NPD_C, 1)[:NP]
        if l < 2:
            table, asrc, adst = _tc_mid(
                acc, table, den, exs, bg[l].reshape(1, H), Wg[l + 1],
                asv[l + 1], adv[l + 1])
        else:
            out = _tc_post(acc, table, den, exs, bg[l].reshape(1, H),
                           W_out1.astype(f32), b_out1.astype(f32).reshape(1, H),
                           W_out2.astype(f32), b_out2.astype(f32).reshape(1, 32))
    return out[:N]


# speedup vs baseline: 32.3390x; 1.0584x over previous
import functools

import jax
import jax.numpy as jnp
from jax import lax
from jax.experimental import pallas as pl
from jax.experimental.pallas import tpu as pltpu
from jax.experimental.pallas import tpu_sc as plsc

N = 10000
E = 320000
F_IN = 128
H = 64

NC = 2
NS = 16
LANES = 16

NP = 10112
ROWW = 32
DUMMY = N
NPD_R = 640
NPD_C = 16

PT = 20480
EP = NS * PT
CH = 512
NCHUNK = PT // CH
IDXR = CH // 128
GROUPS = CH // LANES
ROWS_PER_TILE = NP // NS

_HIGH = jax.lax.Precision.HIGHEST


def _lrelu(t):
    return jnp.where(t > 0, t, 0.2 * t)


BLK = NP // 8


def _dot(a, b):
    return jax.lax.dot_general(a, b, (((1,), (0,)), ((), ())),
                               precision=_HIGH,
                               preferred_element_type=jnp.float32)


def _attn_tail(h, asv, adv, table_ref, asrc_ref, adst_ref):
    asrc_ref[...] = _dot(h, asv)
    adst_ref[...] = _dot(h, adv)
    table_ref[0] = h[:, :ROWW]
    table_ref[1] = h[:, ROWW:]


def _tc_pre_body(x_ref, w_in_ref, b_in_ref, w_g_ref, asv_ref, adv_ref,
                 table_ref, asrc_ref, adst_ref):
    x1 = jnp.maximum(_dot(x_ref[...], w_in_ref[...]) + b_in_ref[...], 0.0)
    h = _dot(x1, w_g_ref[...])
    _attn_tail(h, asv_ref[...], adv_ref[...], table_ref, asrc_ref, adst_ref)


def _combine(acc_ref, tprev_ref, den_ref, exs_ref, b_ref):
    hprev = jnp.concatenate([tprev_ref[0], tprev_ref[1]], axis=1)
    exs = exs_ref[...]
    num = (jnp.concatenate([acc_ref[0], acc_ref[1]], axis=1)
           + exs * hprev)
    den = den_ref[...] + exs + 1e-16
    return num / den + b_ref[...]


def _tc_mid_body(acc_ref, tprev_ref, den_ref, exs_ref, b_ref, w_g_ref,
                 asv_ref, adv_ref, table_ref, asrc_ref, adst_ref):
    y = jnp.maximum(_combine(acc_ref, tprev_ref, den_ref, exs_ref, b_ref),
                    0.0)
    h = _dot(y, w_g_ref[...])
    _attn_tail(h, asv_ref[...], adv_ref[...], table_ref, asrc_ref, adst_ref)


def _tc_shift_body(asrc_ref, adst_ref, cvec_ref, exs_ref):
    asrc = asrc_ref[...]
    adst = adst_ref[...]
    rows = jax.lax.broadcasted_iota(jnp.int32, (NP, 1), 0)
    mask = rows < N
    neg = jnp.float32(-1e30)
    ms = jnp.max(jnp.where(mask, asrc, neg))
    md = jnp.max(jnp.where(mask, adst, neg))
    c = _lrelu(ms + md)
    cvec_ref[...] = jnp.full((1, 128), c, jnp.float32)
    exs_ref[...] = jnp.exp(_lrelu(asrc + adst) - c)


def _tc_post_body(acc_ref, tprev_ref, den_ref, exs_ref, b_ref, w1_ref,
                  b1_ref, w2_ref, b2_ref, out_ref):
    y = _combine(acc_ref, tprev_ref, den_ref, exs_ref, b_ref)
    z = jnp.maximum(_dot(y, w1_ref[...]) + b1_ref[...], 0.0)
    out_ref[...] = _dot(z, w2_ref[...]) + b2_ref[...]


def _full(shape):
    return pl.BlockSpec(shape, lambda i: tuple(0 for _ in shape))


_ROWBLK = pl.BlockSpec((BLK, None), lambda i: (i, 0))
_ROW1 = pl.BlockSpec((BLK, 1), lambda i: (i, 0))
_ACCBLK = pl.BlockSpec((NC, BLK, ROWW), lambda i: (0, i, 0))

_ATTN_OUTS = (
    jax.ShapeDtypeStruct((NC, NP, ROWW), jnp.float32),
    jax.ShapeDtypeStruct((NP, 1), jnp.float32),
    jax.ShapeDtypeStruct((NP, 1), jnp.float32),
)
_ATTN_OUT_SPECS = (_ACCBLK, _ROW1, _ROW1)

_tc_pre = pl.pallas_call(
    _tc_pre_body, out_shape=_ATTN_OUTS, grid=(NP // BLK,),
    in_specs=[pl.BlockSpec((BLK, F_IN), lambda i: (i, 0)),
              _full((F_IN, H)), _full((1, H)), _full((H, H)),
              _full((H, 1)), _full((H, 1))],
    out_specs=_ATTN_OUT_SPECS)
_tc_mid = pl.pallas_call(
    _tc_mid_body, out_shape=_ATTN_OUTS, grid=(NP // BLK,),
    in_specs=[_ACCBLK, _ACCBLK, _ROW1, _ROW1, _full((1, H)), _full((H, H)),
              _full((H, 1)), _full((H, 1))],
    out_specs=_ATTN_OUT_SPECS)
_tc_shift = pl.pallas_call(
    _tc_shift_body,
    out_shape=(jax.ShapeDtypeStruct((1, 128), jnp.float32),
               jax.ShapeDtypeStruct((NP, 1), jnp.float32)))
_tc_post = pl.pallas_call(
    _tc_post_body, out_shape=jax.ShapeDtypeStruct((NP, 32), jnp.float32),
    grid=(NP // BLK,),
    in_specs=[_ACCBLK, _ACCBLK, _ROW1, _ROW1, _full((1, H)), _full((H, H)),
              _full((1, H)), _full((H, 32)), _full((1, 32))],
    out_specs=pl.BlockSpec((BLK, 32), lambda i: (i, 0)))


def _sc_edge_body(src_hbm, dst_hbm, asrc_hbm, adst_hbm, table_hbm, cvec_hbm,
                  acc_hbm, den_hbm,
                  asrc_v, adst_v, cvec_v,
                  sidx_a, sidx_b, didx_a, didx_b, drow_a, drow_b,
                  sdix_a, sdix_b, ex_a, ex_b, exd_a, exd_b,
                  rows_a, rows_b, out_v, dbuf_v, acc_sh, den_sh,
                  sem_ia, sem_ib, sem_ga, sem_gb, sem_sa, sem_sb):
    c = lax.axis_index("c")
    s = lax.axis_index("s")
    coff = jnp.full((LANES,), c * NP, jnp.int32)
    iota = lax.iota(jnp.int32, LANES)
    zero16 = jnp.zeros((LANES,), jnp.float32)
    on_core0 = c == 0

    sidx = (sidx_a, sidx_b)
    didx = (didx_a, didx_b)
    drow = (drow_a, drow_b)
    sdix = (sdix_a, sdix_b)
    exv_ = (ex_a, ex_b)
    exd = (exd_a, exd_b)
    rows = (rows_a, rows_b)
    sem_i = (sem_ia, sem_ib)
    sem_g = (sem_ga, sem_gb)
    sem_s = (sem_sa, sem_sb)

    pltpu.sync_copy(asrc_hbm, asrc_v)
    pltpu.sync_copy(adst_hbm, adst_v)
    pltpu.sync_copy(cvec_hbm, cvec_v)
    cval = cvec_v[pl.ds(0, LANES)]

    def _zero_rows(e, _):
        for k in range(ROWW // LANES):
            rows_a[e, pl.ds(k * LANES, LANES)] = zero16
        exd_a[e, pl.ds(0, LANES)] = zero16
        exd_b[e, pl.ds(0, LANES)] = zero16
        return 0

    lax.fori_loop(0, CH, _zero_rows, 0)
    base = s * ROWS_PER_TILE
    pltpu.sync_copy(rows_a.at[pl.ds(0, CH)], acc_sh.at[pl.ds(base, CH)])
    pltpu.sync_copy(rows_a.at[pl.ds(0, ROWS_PER_TILE - CH)],
                    acc_sh.at[pl.ds(base + CH, ROWS_PER_TILE - CH)])

    @pl.when(on_core0)
    def _():
        def _zero_dbuf(e, _):
            dbuf_v[e, pl.ds(0, LANES)] = zero16
            return 0
        lax.fori_loop(0, NPD_R // NS, _zero_dbuf, 0)
        pltpu.sync_copy(dbuf_v, den_sh.at[pl.ds(s * (NPD_R // NS),
                                                NPD_R // NS)])

    plsc.subcore_barrier()

    idx_base = s * (PT // 128)

    def _issue_idx(k, b):
        pltpu.async_copy(src_hbm.at[pl.ds(idx_base + k * IDXR, IDXR)],
                         sidx[b], sem_i[b])
        pltpu.async_copy(dst_hbm.at[pl.ds(idx_base + k * IDXR, IDXR)],
                         didx[b], sem_i[b])

    def _wait_idx(b):
        pltpu.make_async_copy(src_hbm.at[pl.ds(0, IDXR)], sidx[b],
                              sem_i[b]).wait()
        pltpu.make_async_copy(dst_hbm.at[pl.ds(0, IDXR)], didx[b],
                              sem_i[b]).wait()

    def _alpha_chunk(b):
        def _alpha(g, _):
            jj, gg = g // 8, (g % 8) * LANES
            sv = sidx[b][jj, pl.ds(gg, LANES)]
            dv = didx[b][jj, pl.ds(gg, LANES)]
            a1 = plsc.load_gather(asrc_v, [sv])
            a2 = plsc.load_gather(adst_v, [dv])
            t = a1 + a2
            ex = jnp.exp(_lrelu(t) - cval)
            exv_[b][pl.ds(g * LANES, LANES)] = ex
            sidx[b][jj, pl.ds(gg, LANES)] = sv + coff
            sdix[b][jj, pl.ds(gg, LANES)] = dv

            @pl.when(on_core0)
            def _():
                drow[b][jj, pl.ds(gg, LANES)] = (
                    lax.shift_right_logical(dv, 4))
                ev = g * LANES + iota
                plsc.store_scatter(exd[b], [ev, jnp.bitwise_and(dv, 15)], ex)
            return 0

        lax.fori_loop(0, GROUPS, _alpha, 0, unroll=2)

    def _issue_gather(b):
        for j in range(IDXR):
            pltpu.async_copy(table_hbm.at[sidx[b].at[j]],
                             rows[b].at[pl.ds(j * 128, 128)], sem_g[b])

    def _wait_gather(b):
        pltpu.make_async_copy(table_hbm.at[pl.ds(0, CH)], rows[b],
                              sem_g[b]).wait()

    def _wait_scatter(b):
        pltpu.make_async_copy(rows[b], acc_sh.at[pl.ds(0, CH)],
                              sem_s[b]).wait()

    def _scale_scatter(b):
        def _scale(g, _):
            exv = exv_[b][pl.ds(g * LANES, LANES)]
            for i in range(LANES):
                bi = exv.at[jnp.full((LANES,), i, jnp.int32)].get(
                    mode="promise_in_bounds")
                e = g * LANES + i
                for k in range(ROWW // LANES):
                    sl = pl.ds(k * LANES, LANES)
                    rows[b][e, sl] = rows[b][e, sl] * bi
            return 0

        lax.fori_loop(0, GROUPS, _scale, 0, unroll=2)
        for j in range(IDXR):
            pltpu.async_copy(rows[b].at[pl.ds(j * 128, 128)],
                             acc_sh.at[sdix[b].at[j]], sem_s[b], add=True)

        @pl.when(on_core0)
        def _():
            for j in range(IDXR):
                pltpu.sync_copy(exd[b].at[pl.ds(j * 128, 128)],
                                den_sh.at[drow[b].at[j]], add=True)

            def _rezero(g, _):
                jj, gg = g // 8, (g % 8) * LANES
                dv = didx[b][jj, pl.ds(gg, LANES)]
                ev = g * LANES + iota
                plsc.store_scatter(exd[b], [ev, jnp.bitwise_and(dv, 15)],
                                   zero16)
                return 0

            lax.fori_loop(0, GROUPS, _rezero, 0, unroll=2)

    _issue_idx(0, 0)

    def _pair(base_k, _):
        for b in range(2):
            k = base_k + b
            _wait_idx(b)
            _alpha_chunk(b)

            @pl.when(k > 1)
            def _():
                _wait_scatter(b)

            _issue_gather(b)

            @pl.when(k > 0)
            def _():
                _wait_gather(1 - b)
                _scale_scatter(1 - b)

            @pl.when(k + 1 < NCHUNK)
            def _():
                _issue_idx(k + 1, 1 - b)
        return 0

    lax.fori_loop(0, NCHUNK // 2, lambda i, u: _pair(i * 2, u), 0)
    _wait_gather(1)
    _scale_scatter(1)
    _wait_scatter(0)
    _wait_scatter(1)
    plsc.subcore_barrier()

    pltpu.sync_copy(acc_sh.at[pl.ds(base, ROWS_PER_TILE)], out_v)
    pltpu.sync_copy(out_v, acc_hbm.at[c].at[pl.ds(base, ROWS_PER_TILE)])

    @pl.when(on_core0)
    def _():
        dslice = pl.ds(s * (NPD_R // NS), NPD_R // NS)
        pltpu.sync_copy(den_sh.at[dslice], dbuf_v)
        pltpu.sync_copy(dbuf_v, den_hbm.at[dslice])


@functools.cache
def _get_sc_edge():
  return pl.kernel(
    _sc_edge_body,
    out_type=(jax.ShapeDtypeStruct((NC, NP, ROWW), jnp.float32),
              jax.ShapeDtypeStruct((NPD_R, NPD_C), jnp.float32)),
    mesh=plsc.VectorSubcoreMesh(core_axis_name="c", subcore_axis_name="s",
                                num_cores=NC, num_subcores=NS),
    compiler_params=pltpu.CompilerParams(needs_layout_passes=False,
                                         use_tc_tiling_on_sc=False),
    scratch_types=[
        pltpu.VMEM((NP,), jnp.float32),
        pltpu.VMEM((NP,), jnp.float32),
        pltpu.VMEM((128,), jnp.float32),
        pltpu.VMEM((IDXR, 128), jnp.int32),
        pltpu.VMEM((IDXR, 128), jnp.int32),
        pltpu.VMEM((IDXR, 128), jnp.int32),
        pltpu.VMEM((IDXR, 128), jnp.int32),
        pltpu.VMEM((IDXR, 128), jnp.int32),
        pltpu.VMEM((IDXR, 128), jnp.int32),
        pltpu.VMEM((IDXR, 128), jnp.int32),
        pltpu.VMEM((IDXR, 128), jnp.int32),
        pltpu.VMEM((CH,), jnp.float32),
        pltpu.VMEM((CH,), jnp.float32),
        pltpu.VMEM((CH, NPD_C), jnp.float32),
        pltpu.VMEM((CH, NPD_C), jnp.float32),
        pltpu.VMEM((CH, ROWW), jnp.float32),
        pltpu.VMEM((CH, ROWW), jnp.float32),
        pltpu.VMEM((ROWS_PER_TILE, ROWW), jnp.float32),
        pltpu.VMEM((NPD_R // NS, NPD_C), jnp.float32),
        pltpu.VMEM_SHARED((NP, ROWW), jnp.float32),
        pltpu.VMEM_SHARED((NPD_R, NPD_C), jnp.float32),
        pltpu.SemaphoreType.DMA,
        pltpu.SemaphoreType.DMA,
        pltpu.SemaphoreType.DMA,
        pltpu.SemaphoreType.DMA,
        pltpu.SemaphoreType.DMA,
        pltpu.SemaphoreType.DMA,
    ],
  )


def kernel(x, edge_index, W_in, b_in,
           W_gat0, att_src0, att_dst0, b_gat0,
           W_gat1, att_src1, att_dst1, b_gat1,
           W_gat2, att_src2, att_dst2, b_gat2,
           W_out1, b_out1, W_out2, b_out2):
    f32 = jnp.float32
    x_p = jnp.pad(x.astype(f32), ((0, NP - N), (0, 0)))
    pad = jnp.full((EP - E,), DUMMY, jnp.int32)
    src2 = jnp.concatenate([edge_index[0].astype(jnp.int32), pad]
                           ).reshape(EP // 128, 128)
    dst2 = jnp.concatenate([edge_index[1].astype(jnp.int32), pad]
                           ).reshape(EP // 128, 128)

    asv = [a.reshape(H, 1).astype(f32) for a in (att_src0, att_src1, att_src2)]
    adv = [a.reshape(H, 1).astype(f32) for a in (att_dst0, att_dst1, att_dst2)]
    Wg = [W_gat0.astype(f32), W_gat1.astype(f32), W_gat2.astype(f32)]
    bg = [b_gat0.astype(f32), b_gat1.astype(f32), b_gat2.astype(f32)]

    table, asrc, adst = _tc_pre(
        x_p, W_in.astype(f32), b_in.astype(f32).reshape(1, H), Wg[0],
        asv[0], adv[0])
    for l in range(3):
        cvec, exs = _tc_shift(asrc, adst)
        acc, den = _get_sc_edge()(src2, dst2,
                                  asrc.reshape(NP), adst.reshape(NP),
                                  table.reshape(NC * NP, ROWW),
                                  cvec.reshape(128))
        den = den.reshape(NPD_R * NPD_C, 1)[:NP]
        if l < 2:
            table, asrc, adst = _tc_mid(
                acc, table, den, exs, bg[l].reshape(1, H), Wg[l + 1],
                asv[l + 1], adv[l + 1])
        else:
            out = _tc_post(acc, table, den, exs, bg[l].reshape(1, H),
                           W_out1.astype(f32), b_out1.astype(f32).reshape(1, H),
                           W_out2.astype(f32), b_out2.astype(f32).reshape(1, 32))
    return out[:N]


# speedup vs baseline: 39.2958x; 1.2151x over previous
import functools

import jax
import jax.numpy as jnp
from jax import lax
from jax.experimental import pallas as pl
from jax.experimental.pallas import tpu as pltpu
from jax.experimental.pallas import tpu_sc as plsc

N = 10000
E = 320000
F_IN = 128
H = 64

NC = 2
NS = 16
LANES = 16

NP = 10112
ROWW = 32
DUMMY = N
NPD_R = 640
NPD_C = 16

PT = 20480
EP = NS * PT
CH = 512
NCHUNK = PT // CH
IDXR = CH // 128
GROUPS = CH // LANES
ROWS_PER_TILE = NP // NS

_HIGH = jax.lax.Precision.HIGHEST


def _lrelu(t):
    return jnp.where(t > 0, t, 0.2 * t)


BLK = NP // 8


def _dot(a, b):
    return jax.lax.dot_general(a, b, (((1,), (0,)), ((), ())),
                               precision=_HIGH,
                               preferred_element_type=jnp.float32)


def _attn_tail(h, asv, adv, table_ref, asrc_ref, adst_ref):
    asrc_ref[...] = _dot(h, asv)
    adst_ref[...] = _dot(h, adv)
    table_ref[0] = h[:, :ROWW]
    table_ref[1] = h[:, ROWW:]


def _tc_pre_body(x_ref, w_in_ref, b_in_ref, w_g_ref, asv_ref, adv_ref,
                 table_ref, asrc_ref, adst_ref):
    x1 = jnp.maximum(_dot(x_ref[...], w_in_ref[...]) + b_in_ref[...], 0.0)
    h = _dot(x1, w_g_ref[...])
    _attn_tail(h, asv_ref[...], adv_ref[...], table_ref, asrc_ref, adst_ref)


def _combine(acc_ref, tprev_ref, den_ref, asrc_ref, adst_ref, cvec_ref,
             b_ref):
    hprev = jnp.concatenate([tprev_ref[0], tprev_ref[1]], axis=1)
    c = cvec_ref[0, 0]
    exs = jnp.exp(_lrelu(asrc_ref[...] + adst_ref[...]) - c)
    num = (jnp.concatenate([acc_ref[0], acc_ref[1]], axis=1)
           + exs * hprev)
    den = den_ref[...] + exs + 1e-16
    return num / den + b_ref[...]


def _tc_mid_body(acc_ref, tprev_ref, den_ref, asrc_ref, adst_ref, cvec_ref,
                 b_ref, w_g_ref, asv_ref, adv_ref,
                 table_ref, asrc_o_ref, adst_o_ref):
    y = jnp.maximum(
        _combine(acc_ref, tprev_ref, den_ref, asrc_ref, adst_ref, cvec_ref,
                 b_ref), 0.0)
    h = _dot(y, w_g_ref[...])
    _attn_tail(h, asv_ref[...], adv_ref[...], table_ref, asrc_o_ref,
               adst_o_ref)


def _tc_post_body(acc_ref, tprev_ref, den_ref, asrc_ref, adst_ref, cvec_ref,
                  b_ref, w1_ref, b1_ref, w2_ref, b2_ref, out_ref):
    y = _combine(acc_ref, tprev_ref, den_ref, asrc_ref, adst_ref, cvec_ref,
                 b_ref)
    z = jnp.maximum(_dot(y, w1_ref[...]) + b1_ref[...], 0.0)
    out_ref[...] = _dot(z, w2_ref[...]) + b2_ref[...]


def _full(shape):
    return pl.BlockSpec(shape, lambda i: tuple(0 for _ in shape))


_ROWBLK = pl.BlockSpec((BLK, None), lambda i: (i, 0))
_ROW1 = pl.BlockSpec((BLK, 1), lambda i: (i, 0))
_ACCBLK = pl.BlockSpec((NC, BLK, ROWW), lambda i: (0, i, 0))

_ATTN_OUTS = (
    jax.ShapeDtypeStruct((NC, NP, ROWW), jnp.float32),
    jax.ShapeDtypeStruct((NP, 1), jnp.float32),
    jax.ShapeDtypeStruct((NP, 1), jnp.float32),
)
_ATTN_OUT_SPECS = (_ACCBLK, _ROW1, _ROW1)

_tc_pre = pl.pallas_call(
    _tc_pre_body, out_shape=_ATTN_OUTS, grid=(NP // BLK,),
    in_specs=[pl.BlockSpec((BLK, F_IN), lambda i: (i, 0)),
              _full((F_IN, H)), _full((1, H)), _full((H, H)),
              _full((H, 1)), _full((H, 1))],
    out_specs=_ATTN_OUT_SPECS)
_tc_mid = pl.pallas_call(
    _tc_mid_body, out_shape=_ATTN_OUTS, grid=(NP // BLK,),
    in_specs=[_ACCBLK, _ACCBLK, _ROW1, _ROW1, _ROW1, _full((1, 128)),
              _full((1, H)), _full((H, H)), _full((H, 1)), _full((H, 1))],
    out_specs=_ATTN_OUT_SPECS)
_tc_post = pl.pallas_call(
    _tc_post_body, out_shape=jax.ShapeDtypeStruct((NP, 32), jnp.float32),
    grid=(NP // BLK,),
    in_specs=[_ACCBLK, _ACCBLK, _ROW1, _ROW1, _ROW1, _full((1, 128)),
              _full((1, H)), _full((H, H)),
              _full((1, H)), _full((H, 32)), _full((1, 32))],
    out_specs=pl.BlockSpec((BLK, 32), lambda i: (i, 0)))


def _sc_edge_body(src_hbm, dst_hbm, asrc_hbm, adst_hbm, table_hbm,
                  acc_hbm, den_hbm, cvec_hbm,
                  asrc_v, adst_v, cvec_v,
                  sidx_a, sidx_b, didx_a, didx_b, drow_a, drow_b,
                  sdix_a, sdix_b, ex_a, ex_b, exd_a, exd_b,
                  rows_a, rows_b, out_v, dbuf_v, acc_sh, den_sh,
                  sem_ia, sem_ib, sem_ga, sem_gb, sem_sa, sem_sb):
    c = lax.axis_index("c")
    s = lax.axis_index("s")
    coff = jnp.full((LANES,), c * NP, jnp.int32)
    iota = lax.iota(jnp.int32, LANES)
    zero16 = jnp.zeros((LANES,), jnp.float32)
    on_core0 = c == 0

    sidx = (sidx_a, sidx_b)
    didx = (didx_a, didx_b)
    drow = (drow_a, drow_b)
    sdix = (sdix_a, sdix_b)
    exv_ = (ex_a, ex_b)
    exd = (exd_a, exd_b)
    rows = (rows_a, rows_b)
    sem_i = (sem_ia, sem_ib)
    sem_g = (sem_ga, sem_gb)
    sem_s = (sem_sa, sem_sb)

    pltpu.sync_copy(asrc_hbm, asrc_v)
    pltpu.sync_copy(adst_hbm, adst_v)
    neg = jnp.full((LANES,), -1e30, jnp.float32)
    ms = lax.fori_loop(
        0, NP // LANES,
        lambda g, m: jnp.maximum(m, asrc_v[pl.ds(g * LANES, LANES)]),
        neg, unroll=4)
    md = lax.fori_loop(
        0, NP // LANES,
        lambda g, m: jnp.maximum(m, adst_v[pl.ds(g * LANES, LANES)]),
        neg, unroll=4)
    cscal = _lrelu(jnp.max(ms) + jnp.max(md))
    cval = jnp.full((LANES,), cscal, jnp.float32)

    @pl.when(on_core0 & (s == 0))
    def _():
        for k in range(128 // LANES):
            cvec_v[pl.ds(k * LANES, LANES)] = cval
        pltpu.sync_copy(cvec_v, cvec_hbm)

    def _zero_rows(e, _):
        for k in range(ROWW // LANES):
            rows_a[e, pl.ds(k * LANES, LANES)] = zero16
        exd_a[e, pl.ds(0, LANES)] = zero16
        exd_b[e, pl.ds(0, LANES)] = zero16
        return 0

    lax.fori_loop(0, CH, _zero_rows, 0)
    base = s * ROWS_PER_TILE
    pltpu.sync_copy(rows_a.at[pl.ds(0, CH)], acc_sh.at[pl.ds(base, CH)])
    pltpu.sync_copy(rows_a.at[pl.ds(0, ROWS_PER_TILE - CH)],
                    acc_sh.at[pl.ds(base + CH, ROWS_PER_TILE - CH)])

    @pl.when(on_core0)
    def _():
        def _zero_dbuf(e, _):
            dbuf_v[e, pl.ds(0, LANES)] = zero16
            return 0
        lax.fori_loop(0, NPD_R // NS, _zero_dbuf, 0)
        pltpu.sync_copy(dbuf_v, den_sh.at[pl.ds(s * (NPD_R // NS),
                                                NPD_R // NS)])

    plsc.subcore_barrier()

    idx_base = s * (PT // 128)

    def _issue_idx(k, b):
        pltpu.async_copy(src_hbm.at[pl.ds(idx_base + k * IDXR, IDXR)],
                         sidx[b], sem_i[b])
        pltpu.async_copy(dst_hbm.at[pl.ds(idx_base + k * IDXR, IDXR)],
                         didx[b], sem_i[b])

    def _wait_idx(b):
        pltpu.make_async_copy(src_hbm.at[pl.ds(0, IDXR)], sidx[b],
                              sem_i[b]).wait()
        pltpu.make_async_copy(dst_hbm.at[pl.ds(0, IDXR)], didx[b],
                              sem_i[b]).wait()

    def _alpha_chunk(b):
        def _alpha(g, _):
            jj, gg = g // 8, (g % 8) * LANES
            sv = sidx[b][jj, pl.ds(gg, LANES)]
            dv = didx[b][jj, pl.ds(gg, LANES)]
            a1 = plsc.load_gather(asrc_v, [sv])
            a2 = plsc.load_gather(adst_v, [dv])
            t = a1 + a2
            ex = jnp.exp(_lrelu(t) - cval)
            exv_[b][pl.ds(g * LANES, LANES)] = ex
            sidx[b][jj, pl.ds(gg, LANES)] = sv + coff
            sdix[b][jj, pl.ds(gg, LANES)] = dv

            @pl.when(on_core0)
            def _():
                drow[b][jj, pl.ds(gg, LANES)] = (
                    lax.shift_right_logical(dv, 4))
                ev = g * LANES + iota
                plsc.store_scatter(exd[b], [ev, jnp.bitwise_and(dv, 15)], ex)
            return 0

        lax.fori_loop(0, GROUPS, _alpha, 0, unroll=2)

    def _issue_gather(b):
        for j in range(IDXR):
            pltpu.async_copy(table_hbm.at[sidx[b].at[j]],
                             rows[b].at[pl.ds(j * 128, 128)], sem_g[b])

    def _wait_gather(b):
        pltpu.make_async_copy(table_hbm.at[pl.ds(0, CH)], rows[b],
                              sem_g[b]).wait()

    def _wait_scatter(b):
        pltpu.make_async_copy(rows[b], acc_sh.at[pl.ds(0, CH)],
                              sem_s[b]).wait()

    def _scale_scatter(b):
        def _scale(g, _):
            exv = exv_[b][pl.ds(g * LANES, LANES)]
            for i in range(LANES):
                bi = exv.at[jnp.full((LANES,), i, jnp.int32)].get(
                    mode="promise_in_bounds")
                e = g * LANES + i
                for k in range(ROWW // LANES):
                    sl = pl.ds(k * LANES, LANES)
                    rows[b][e, sl] = rows[b][e, sl] * bi
            return 0

        lax.fori_loop(0, GROUPS, _scale, 0, unroll=4)
        for j in range(IDXR):
            pltpu.async_copy(rows[b].at[pl.ds(j * 128, 128)],
                             acc_sh.at[sdix[b].at[j]], sem_s[b], add=True)

        @pl.when(on_core0)
        def _():
            for j in range(IDXR):
                pltpu.sync_copy(exd[b].at[pl.ds(j * 128, 128)],
                                den_sh.at[drow[b].at[j]], add=True)

            def _rezero(g, _):
                jj, gg = g // 8, (g % 8) * LANES
                dv = didx[b][jj, pl.ds(gg, LANES)]
                ev = g * LANES + iota
                plsc.store_scatter(exd[b], [ev, jnp.bitwise_and(dv, 15)],
                                   zero16)
                return 0

            lax.fori_loop(0, GROUPS, _rezero, 0, unroll=2)

    _issue_idx(0, 0)

    def _pair(base_k, _):
        for b in range(2):
            k = base_k + b
            _wait_idx(b)
            _alpha_chunk(b)

            @pl.when(k > 1)
            def _():
                _wait_scatter(b)

            _issue_gather(b)

            @pl.when(k > 0)
            def _():
                _wait_gather(1 - b)
                _scale_scatter(1 - b)

            @pl.when(k + 1 < NCHUNK)
            def _():
                _issue_idx(k + 1, 1 - b)
        return 0

    lax.fori_loop(0, NCHUNK // 2, lambda i, u: _pair(i * 2, u), 0)
    _wait_gather(1)
    _scale_scatter(1)
    _wait_scatter(0)
    _wait_scatter(1)
    plsc.subcore_barrier()

    pltpu.sync_copy(acc_sh.at[pl.ds(base, ROWS_PER_TILE)], out_v)
    pltpu.sync_copy(out_v, acc_hbm.at[c].at[pl.ds(base, ROWS_PER_TILE)])

    @pl.when(on_core0)
    def _():
        dslice = pl.ds(s * (NPD_R // NS), NPD_R // NS)
        pltpu.sync_copy(den_sh.at[dslice], dbuf_v)
        pltpu.sync_copy(dbuf_v, den_hbm.at[dslice])


@functools.cache
def _get_sc_edge():
  return pl.kernel(
    _sc_edge_body,
    out_type=(jax.ShapeDtypeStruct((NC, NP, ROWW), jnp.float32),
              jax.ShapeDtypeStruct((NPD_R, NPD_C), jnp.float32),
              jax.ShapeDtypeStruct((128,), jnp.float32)),
    mesh=plsc.VectorSubcoreMesh(core_axis_name="c", subcore_axis_name="s",
                                num_cores=NC, num_subcores=NS),
    compiler_params=pltpu.CompilerParams(needs_layout_passes=False,
                                         use_tc_tiling_on_sc=False),
    scratch_types=[
        pltpu.VMEM((NP,), jnp.float32),
        pltpu.VMEM((NP,), jnp.float32),
        pltpu.VMEM((128,), jnp.float32),
        pltpu.VMEM((IDXR, 128), jnp.int32),
        pltpu.VMEM((IDXR, 128), jnp.int32),
        pltpu.VMEM((IDXR, 128), jnp.int32),
        pltpu.VMEM((IDXR, 128), jnp.int32),
        pltpu.VMEM((IDXR, 128), jnp.int32),
        pltpu.VMEM((IDXR, 128), jnp.int32),
        pltpu.VMEM((IDXR, 128), jnp.int32),
        pltpu.VMEM((IDXR, 128), jnp.int32),
        pltpu.VMEM((CH,), jnp.float32),
        pltpu.VMEM((CH,), jnp.float32),
        pltpu.VMEM((CH, NPD_C), jnp.float32),
        pltpu.VMEM((CH, NPD_C), jnp.float32),
        pltpu.VMEM((CH, ROWW), jnp.float32),
        pltpu.VMEM((CH, ROWW), jnp.float32),
        pltpu.VMEM((ROWS_PER_TILE, ROWW), jnp.float32),
        pltpu.VMEM((NPD_R // NS, NPD_C), jnp.float32),
        pltpu.VMEM_SHARED((NP, ROWW), jnp.float32),
        pltpu.VMEM_SHARED((NPD_R, NPD_C), jnp.float32),
        pltpu.SemaphoreType.DMA,
        pltpu.SemaphoreType.DMA,
        pltpu.SemaphoreType.DMA,
        pltpu.SemaphoreType.DMA,
        pltpu.SemaphoreType.DMA,
        pltpu.SemaphoreType.DMA,
    ],
  )


def kernel(x, edge_index, W_in, b_in,
           W_gat0, att_src0, att_dst0, b_gat0,
           W_gat1, att_src1, att_dst1, b_gat1,
           W_gat2, att_src2, att_dst2, b_gat2,
           W_out1, b_out1, W_out2, b_out2):
    f32 = jnp.float32
    x_p = jnp.pad(x.astype(f32), ((0, NP - N), (0, 0)))
    pad = jnp.full((EP - E,), DUMMY, jnp.int32)
    src2 = jnp.concatenate([edge_index[0].astype(jnp.int32), pad]
                           ).reshape(EP // 128, 128)
    dst2 = jnp.concatenate([edge_index[1].astype(jnp.int32), pad]
                           ).reshape(EP // 128, 128)

    asv = [a.reshape(H, 1).astype(f32) for a in (att_src0, att_src1, att_src2)]
    adv = [a.reshape(H, 1).astype(f32) for a in (att_dst0, att_dst1, att_dst2)]
    Wg = [W_gat0.astype(f32), W_gat1.astype(f32), W_gat2.astype(f32)]
    bg = [b_gat0.astype(f32), b_gat1.astype(f32), b_gat2.astype(f32)]

    table, asrc, adst = _tc_pre(
        x_p, W_in.astype(f32), b_in.astype(f32).reshape(1, H), Wg[0],
        asv[0], adv[0])
    for l in range(3):
        acc, den, cvec = _get_sc_edge()(src2, dst2,
                                        asrc.reshape(NP), adst.reshape(NP),
                                        table.reshape(NC * NP, ROWW))
        den = den.reshape(NPD_R * NPD_C, 1)[:NP]
        cvec = cvec.reshape(1, 128)
        if l < 2:
            table, asrc, adst = _tc_mid(
                acc, table, den, asrc, adst, cvec,
                bg[l].reshape(1, H), Wg[l + 1],
                asv[l + 1], adv[l + 1])
        else:
            out = _tc_post(acc, table, den, asrc, adst, cvec,
                           bg[l].reshape(1, H),
                           W_out1.astype(f32), b_out1.astype(f32).reshape(1, H),
                           W_out2.astype(f32), b_out2.astype(f32).reshape(1, 32))
    return out[:N]


# speedup vs baseline: 40.0796x; 1.0199x over previous
import functools

import jax
import jax.numpy as jnp
from jax import lax
from jax.experimental import pallas as pl
from jax.experimental.pallas import tpu as pltpu
from jax.experimental.pallas import tpu_sc as plsc

N = 10000
E = 320000
F_IN = 128
H = 64

NC = 2
NS = 16
LANES = 16

NP = 10112
ROWW = 32
DUMMY = N
NPD_R = 640
NPD_C = 16
NACC = 10008

PT = 20480
EP = NS * PT
CH = 640
NCHUNK = PT // CH
IDXR = CH // 128
GROUPS = CH // LANES
ROWS_PER_TILE = NP // NS

_HIGH = jax.lax.Precision.HIGHEST


def _lrelu(t):
    return jnp.where(t > 0, t, 0.2 * t)


BLK = NP // 8


def _dot(a, b):
    return jax.lax.dot_general(a, b, (((1,), (0,)), ((), ())),
                               precision=_HIGH,
                               preferred_element_type=jnp.float32)


def _attn_tail(h, asv, adv, table_ref, asrc_ref, adst_ref):
    asrc_ref[...] = _dot(h, asv)
    adst_ref[...] = _dot(h, adv)
    table_ref[0] = h[:, :ROWW]
    table_ref[1] = h[:, ROWW:]


def _tc_pre_body(x_ref, w_in_ref, b_in_ref, w_g_ref, asv_ref, adv_ref,
                 table_ref, asrc_ref, adst_ref):
    x1 = jnp.maximum(_dot(x_ref[...], w_in_ref[...]) + b_in_ref[...], 0.0)
    h = _dot(x1, w_g_ref[...])
    _attn_tail(h, asv_ref[...], adv_ref[...], table_ref, asrc_ref, adst_ref)


def _combine(acc_ref, tprev_ref, den_ref, asrc_ref, adst_ref, cvec_ref,
             b_ref):
    hprev = jnp.concatenate([tprev_ref[0], tprev_ref[1]], axis=1)
    c = cvec_ref[0, 0]
    exs = jnp.exp(_lrelu(asrc_ref[...] + adst_ref[...]) - c)
    num = (jnp.concatenate([acc_ref[0], acc_ref[1]], axis=1)
           + exs * hprev)
    den = den_ref[...] + exs + 1e-16
    return num / den + b_ref[...]


def _tc_mid_body(acc_ref, tprev_ref, den_ref, asrc_ref, adst_ref, cvec_ref,
                 b_ref, w_g_ref, asv_ref, adv_ref,
                 table_ref, asrc_o_ref, adst_o_ref):
    y = jnp.maximum(
        _combine(acc_ref, tprev_ref, den_ref, asrc_ref, adst_ref, cvec_ref,
                 b_ref), 0.0)
    h = _dot(y, w_g_ref[...])
    _attn_tail(h, asv_ref[...], adv_ref[...], table_ref, asrc_o_ref,
               adst_o_ref)


def _tc_post_body(acc_ref, tprev_ref, den_ref, asrc_ref, adst_ref, cvec_ref,
                  b_ref, w1_ref, b1_ref, w2_ref, b2_ref, out_ref):
    y = _combine(acc_ref, tprev_ref, den_ref, asrc_ref, adst_ref, cvec_ref,
                 b_ref)
    z = jnp.maximum(_dot(y, w1_ref[...]) + b1_ref[...], 0.0)
    out_ref[...] = _dot(z, w2_ref[...]) + b2_ref[...]


def _full(shape):
    return pl.BlockSpec(shape, lambda i: tuple(0 for _ in shape))


_ROWBLK = pl.BlockSpec((BLK, None), lambda i: (i, 0))
_ROW1 = pl.BlockSpec((BLK, 1), lambda i: (i, 0))
_ACCBLK = pl.BlockSpec((NC, BLK, ROWW), lambda i: (0, i, 0))

_ATTN_OUTS = (
    jax.ShapeDtypeStruct((NC, NP, ROWW), jnp.float32),
    jax.ShapeDtypeStruct((NP, 1), jnp.float32),
    jax.ShapeDtypeStruct((NP, 1), jnp.float32),
)
_ATTN_OUT_SPECS = (_ACCBLK, _ROW1, _ROW1)

_tc_pre = pl.pallas_call(
    _tc_pre_body, out_shape=_ATTN_OUTS, grid=(NP // BLK,),
    in_specs=[pl.BlockSpec((BLK, F_IN), lambda i: (i, 0)),
              _full((F_IN, H)), _full((1, H)), _full((H, H)),
              _full((H, 1)), _full((H, 1))],
    out_specs=_ATTN_OUT_SPECS)
_tc_mid = pl.pallas_call(
    _tc_mid_body, out_shape=_ATTN_OUTS, grid=(NP // BLK,),
    in_specs=[_ACCBLK, _ACCBLK, _ROW1, _ROW1, _ROW1, _full((1, 128)),
              _full((1, H)), _full((H, H)), _full((H, 1)), _full((H, 1))],
    out_specs=_ATTN_OUT_SPECS)
_tc_post = pl.pallas_call(
    _tc_post_body, out_shape=jax.ShapeDtypeStruct((NP, 32), jnp.float32),
    grid=(NP // BLK,),
    in_specs=[_ACCBLK, _ACCBLK, _ROW1, _ROW1, _ROW1, _full((1, 128)),
              _full((1, H)), _full((H, H)),
              _full((1, H)), _full((H, 32)), _full((1, 32))],
    out_specs=pl.BlockSpec((BLK, 32), lambda i: (i, 0)))


def _sc_edge_body(src_hbm, dst_hbm, asrc_hbm, adst_hbm, table_hbm,
                  acc_hbm, den_hbm, cvec_hbm,
                  asrc_v, adst_v, cvec_v,
                  sidx_a, sidx_b, didx_a, didx_b, drow_a, drow_b,
                  sdix_a, sdix_b, dcol_a, dcol_b, ex_a, ex_b, exd_a, exd_b,
                  rows_a, rows_b, out_v, dbuf_v, acc_sh, den_sh,
                  sem_i0, sem_ga, sem_gb, sem_s0, sem_d0):
    c = lax.axis_index("c")
    s = lax.axis_index("s")
    coff = jnp.full((LANES,), c * NP, jnp.int32)
    iota = lax.iota(jnp.int32, LANES)
    zero16 = jnp.zeros((LANES,), jnp.float32)
    on_core0 = c == 0

    sidx = (sidx_a, sidx_b)
    didx = (didx_a, didx_b)
    drow = (drow_a, drow_b)
    sdix = (sdix_a, sdix_b)
    dcol = (dcol_a, dcol_b)
    exv_ = (ex_a, ex_b)
    exd = (exd_a, exd_b)
    rows = (rows_a, rows_b)
    sem_i = (sem_i0, sem_i0)
    sem_g = (sem_ga, sem_gb)
    sem_s = (sem_s0, sem_s0)
    sem_d = (sem_d0, sem_d0)

    pltpu.sync_copy(asrc_hbm, asrc_v)
    pltpu.sync_copy(adst_hbm, adst_v)
    neg = jnp.full((LANES,), -1e30, jnp.float32)
    ms = lax.fori_loop(
        0, NP // LANES,
        lambda g, m: jnp.maximum(m, asrc_v[pl.ds(g * LANES, LANES)]),
        neg, unroll=4)
    md = lax.fori_loop(
        0, NP // LANES,
        lambda g, m: jnp.maximum(m, adst_v[pl.ds(g * LANES, LANES)]),
        neg, unroll=4)
    cscal = _lrelu(jnp.max(ms) + jnp.max(md))
    cval = jnp.full((LANES,), cscal, jnp.float32)

    @pl.when(on_core0 & (s == 0))
    def _():
        for k in range(128 // LANES):
            cvec_v[pl.ds(k * LANES, LANES)] = cval
        pltpu.sync_copy(cvec_v, cvec_hbm)

    def _zero_rows(e, _):
        for k in range(ROWW // LANES):
            rows_a[e, pl.ds(k * LANES, LANES)] = zero16
        exd_a[e, pl.ds(0, LANES)] = zero16
        exd_b[e, pl.ds(0, LANES)] = zero16
        return 0

    lax.fori_loop(0, CH, _zero_rows, 0)
    base = s * ROWS_PER_TILE
    last = NACC - 15 * ROWS_PER_TILE

    @pl.when(s < 15)
    def _():
        pltpu.sync_copy(rows_a.at[pl.ds(0, ROWS_PER_TILE)],
                        acc_sh.at[pl.ds(base, ROWS_PER_TILE)])

    @pl.when(s == 15)
    def _():
        pltpu.sync_copy(rows_a.at[pl.ds(0, last)],
                        acc_sh.at[pl.ds(base, last)])

    @pl.when(on_core0)
    def _():
        def _zero_dbuf(e, _):
            dbuf_v[e, pl.ds(0, LANES)] = zero16
            return 0
        lax.fori_loop(0, NPD_R // NS, _zero_dbuf, 0)
        pltpu.sync_copy(dbuf_v, den_sh.at[pl.ds(s * (NPD_R // NS),
                                                NPD_R // NS)])

    plsc.subcore_barrier()

    idx_base = s * (PT // 128)

    def _issue_idx(k, b):
        pltpu.async_copy(src_hbm.at[pl.ds(idx_base + k * IDXR, IDXR)],
                         sidx[b], sem_i[b])
        pltpu.async_copy(dst_hbm.at[pl.ds(idx_base + k * IDXR, IDXR)],
                         didx[b], sem_i[b])

    def _wait_idx(b):
        pltpu.make_async_copy(src_hbm.at[pl.ds(0, IDXR)], sidx[b],
                              sem_i[b]).wait()
        pltpu.make_async_copy(dst_hbm.at[pl.ds(0, IDXR)], didx[b],
                              sem_i[b]).wait()

    def _alpha_chunk(b):
        def _alpha(g, _):
            jj, gg = g // 8, (g % 8) * LANES
            sv = sidx[b][jj, pl.ds(gg, LANES)]
            dv = didx[b][jj, pl.ds(gg, LANES)]
            a1 = plsc.load_gather(asrc_v, [sv])
            a2 = plsc.load_gather(adst_v, [dv])
            t = a1 + a2
            ex = jnp.exp(_lrelu(t) - cval)
            exv_[b][pl.ds(g * LANES, LANES)] = ex
            sidx[b][jj, pl.ds(gg, LANES)] = sv + coff
            sdix[b][jj, pl.ds(gg, LANES)] = dv

            @pl.when(on_core0)
            def _():
                drow[b][jj, pl.ds(gg, LANES)] = (
                    lax.shift_right_logical(dv, 4))
                dc = jnp.bitwise_and(dv, 15)
                dcol[b][jj, pl.ds(gg, LANES)] = dc
                ev = g * LANES + iota
                plsc.store_scatter(exd[b], [ev, dc], ex)
            return 0

        lax.fori_loop(0, GROUPS, _alpha, 0, unroll=2)

    def _issue_gather(b):
        for j in range(IDXR):
            pltpu.async_copy(table_hbm.at[sidx[b].at[j]],
                             rows[b].at[pl.ds(j * 128, 128)], sem_g[b])

    def _wait_gather(b):
        pltpu.make_async_copy(table_hbm.at[pl.ds(0, CH)], rows[b],
                              sem_g[b]).wait()

    def _wait_scatter(b):
        pltpu.make_async_copy(rows[b], acc_sh.at[pl.ds(0, CH)],
                              sem_s[b]).wait()

    def _scale_scatter(b):
        def _scale(g, _):
            exv = exv_[b][pl.ds(g * LANES, LANES)]
            for i in range(LANES):
                bi = exv.at[jnp.full((LANES,), i, jnp.int32)].get(
                    mode="promise_in_bounds")
                e = g * LANES + i
                for k in range(ROWW // LANES):
                    sl = pl.ds(k * LANES, LANES)
                    rows[b][e, sl] = rows[b][e, sl] * bi
            return 0

        lax.fori_loop(0, GROUPS, _scale, 0, unroll=4)
        for j in range(IDXR):
            pltpu.async_copy(rows[b].at[pl.ds(j * 128, 128)],
                             acc_sh.at[sdix[b].at[j]], sem_s[b], add=True)

        @pl.when(on_core0)
        def _():
            for j in range(IDXR):
                pltpu.async_copy(exd[b].at[pl.ds(j * 128, 128)],
                                 den_sh.at[drow[b].at[j]], sem_d[b],
                                 add=True)

    def _wait_den_rezero(b):
        @pl.when(on_core0)
        def _():
            pltpu.make_async_copy(exd[b], den_sh, sem_d[b]).wait()

            def _rezero(g, _):
                jj, gg = g // 8, (g % 8) * LANES
                dc = dcol[b][jj, pl.ds(gg, LANES)]
                ev = g * LANES + iota
                plsc.store_scatter(exd[b], [ev, dc], zero16)
                return 0

            lax.fori_loop(0, GROUPS, _rezero, 0, unroll=2)

    _issue_idx(0, 0)

    def _pair(base_k, _):
        for b in range(2):
            k = base_k + b
            _wait_idx(b)

            @pl.when(k > 1)
            def _():
                _wait_den_rezero(b)

            _alpha_chunk(b)

            @pl.when(k > 1)
            def _():
                _wait_scatter(b)

            _issue_gather(b)

            @pl.when(k > 0)
            def _():
                _wait_gather(1 - b)
                _scale_scatter(1 - b)

            @pl.when(k + 1 < NCHUNK)
            def _():
                _issue_idx(k + 1, 1 - b)
        return 0

    lax.fori_loop(0, NCHUNK // 2, lambda i, u: _pair(i * 2, u), 0)
    _wait_gather(1)
    _scale_scatter(1)
    _wait_scatter(0)
    _wait_scatter(1)

    @pl.when(on_core0)
    def _():
        pltpu.make_async_copy(exd_a, den_sh, sem_d0).wait()
        pltpu.make_async_copy(exd_b, den_sh, sem_d0).wait()

    plsc.subcore_barrier()

    @pl.when(s < 15)
    def _():
        pltpu.sync_copy(acc_sh.at[pl.ds(base, ROWS_PER_TILE)], out_v)
        pltpu.sync_copy(out_v, acc_hbm.at[c].at[pl.ds(base, ROWS_PER_TILE)])

    @pl.when(s == 15)
    def _():
        pltpu.sync_copy(acc_sh.at[pl.ds(base, last)],
                        out_v.at[pl.ds(0, last)])
        pltpu.sync_copy(out_v.at[pl.ds(0, last)],
                        acc_hbm.at[c].at[pl.ds(base, last)])

    @pl.when(on_core0)
    def _():
        dslice = pl.ds(s * (NPD_R // NS), NPD_R // NS)
        pltpu.sync_copy(den_sh.at[dslice], dbuf_v)
        pltpu.sync_copy(dbuf_v, den_hbm.at[dslice])


@functools.cache
def _get_sc_edge():
  return pl.kernel(
    _sc_edge_body,
    out_type=(jax.ShapeDtypeStruct((NC, NACC, ROWW), jnp.float32),
              jax.ShapeDtypeStruct((NPD_R, NPD_C), jnp.float32),
              jax.ShapeDtypeStruct((128,), jnp.float32)),
    mesh=plsc.VectorSubcoreMesh(core_axis_name="c", subcore_axis_name="s",
                                num_cores=NC, num_subcores=NS),
    compiler_params=pltpu.CompilerParams(needs_layout_passes=False,
                                         use_tc_tiling_on_sc=False),
    scratch_types=[
        pltpu.VMEM((NP,), jnp.float32),
        pltpu.VMEM((NP,), jnp.float32),
        pltpu.VMEM((128,), jnp.float32),
        pltpu.VMEM((IDXR, 128), jnp.int32),
        pltpu.VMEM((IDXR, 128), jnp.int32),
        pltpu.VMEM((IDXR, 128), jnp.int32),
        pltpu.VMEM((IDXR, 128), jnp.int32),
        pltpu.VMEM((IDXR, 128), jnp.int32),
        pltpu.VMEM((IDXR, 128), jnp.int32),
        pltpu.VMEM((IDXR, 128), jnp.int32),
        pltpu.VMEM((IDXR, 128), jnp.int32),
        pltpu.VMEM((IDXR, 128), jnp.int32),
        pltpu.VMEM((IDXR, 128), jnp.int32),
        pltpu.VMEM((CH,), jnp.float32),
        pltpu.VMEM((CH,), jnp.float32),
        pltpu.VMEM((CH, NPD_C), jnp.float32),
        pltpu.VMEM((CH, NPD_C), jnp.float32),
        pltpu.VMEM((CH, ROWW), jnp.float32),
        pltpu.VMEM((CH, ROWW), jnp.float32),
        pltpu.VMEM((ROWS_PER_TILE, ROWW), jnp.float32),
        pltpu.VMEM((NPD_R // NS, NPD_C), jnp.float32),
        pltpu.VMEM_SHARED((NACC, ROWW), jnp.float32),
        pltpu.VMEM_SHARED((NPD_R, NPD_C), jnp.float32),
        pltpu.SemaphoreType.DMA,
        pltpu.SemaphoreType.DMA,
        pltpu.SemaphoreType.DMA,
        pltpu.SemaphoreType.DMA,
        pltpu.SemaphoreType.DMA,
    ],
  )


def kernel(x, edge_index, W_in, b_in,
           W_gat0, att_src0, att_dst0, b_gat0,
           W_gat1, att_src1, att_dst1, b_gat1,
           W_gat2, att_src2, att_dst2, b_gat2,
           W_out1, b_out1, W_out2, b_out2):
    f32 = jnp.float32
    x_p = jnp.pad(x.astype(f32), ((0, NP - N), (0, 0)))
    pad = jnp.full((EP - E,), DUMMY, jnp.int32)
    src2 = jnp.concatenate([edge_index[0].astype(jnp.int32), pad]
                           ).reshape(EP // 128, 128)
    dst2 = jnp.concatenate([edge_index[1].astype(jnp.int32), pad]
                           ).reshape(EP // 128, 128)

    asv = [a.reshape(H, 1).astype(f32) for a in (att_src0, att_src1, att_src2)]
    adv = [a.reshape(H, 1).astype(f32) for a in (att_dst0, att_dst1, att_dst2)]
    Wg = [W_gat0.astype(f32), W_gat1.astype(f32), W_gat2.astype(f32)]
    bg = [b_gat0.astype(f32), b_gat1.astype(f32), b_gat2.astype(f32)]

    table, asrc, adst = _tc_pre(
        x_p, W_in.astype(f32), b_in.astype(f32).reshape(1, H), Wg[0],
        asv[0], adv[0])
    for l in range(3):
        acc, den, cvec = _get_sc_edge()(src2, dst2,
                                        asrc.reshape(NP), adst.reshape(NP),
                                        table.reshape(NC * NP, ROWW))
        acc = jnp.pad(acc, ((0, 0), (0, NP - NACC), (0, 0)))
        den = den.reshape(NPD_R * NPD_C, 1)[:NP]
        cvec = cvec.reshape(1, 128)
        if l < 2:
            table, asrc, adst = _tc_mid(
                acc, table, den, asrc, adst, cvec,
                bg[l].reshape(1, H), Wg[l + 1],
                asv[l + 1], adv[l + 1])
        else:
            out = _tc_post(acc, table, den, asrc, adst, cvec,
                           bg[l].reshape(1, H),
                           W_out1.astype(f32), b_out1.astype(f32).reshape(1, H),
                           W_out2.astype(f32), b_out2.astype(f32).reshape(1, 32))
    return out[:N]
